# baseline (speedup 1.0000x reference)
import functools

import jax
import jax.numpy as jnp
from jax import lax
from jax.experimental import pallas as pl
from jax.experimental.pallas import tpu as pltpu
from jax.experimental.pallas import tpu_sc as plsc

H = 1_000_000
ALPHA = 0.01
L = 16
IDX_CHUNK = 128


def _make_sc_kernel(batch, num_workers, num_cores):
    n_per_w = batch // num_workers
    n_chunks = n_per_w // IDX_CHUNK
    mesh = plsc.VectorSubcoreMesh(core_axis_name="c", subcore_axis_name="s",
                                  num_cores=num_cores)

    @functools.partial(
        pl.kernel,
        mesh=mesh,
        out_type=jax.ShapeDtypeStruct((batch,), jnp.float32),
        scratch_types=[
            pltpu.VMEM((n_per_w,), jnp.int32),
            pltpu.VMEM((n_chunks, IDX_CHUNK), jnp.int32),
            pltpu.VMEM((n_per_w,), jnp.float32),
            pltpu.VMEM((n_per_w,), jnp.float32),
            pltpu.VMEM((L,), jnp.float32),
            pltpu.VMEM((n_per_w,), jnp.float32),
            pltpu.SemaphoreType.DMA,
            pltpu.SemaphoreType.DMA,
            pltpu.SemaphoreType.DMA,
        ],
    )
    def sc_kernel(ids_hbm, a_hbm, b_hbm, step_hbm, out_hbm,
                  ids_v, q_v, a_v, b_v, step_v, o_v, sem_a, sem_b, sem_in):
        wid = lax.axis_index("s") * num_cores + lax.axis_index("c")
        base = wid * n_per_w

        cp_ids = pltpu.async_copy(ids_hbm.at[pl.ds(base, n_per_w)], ids_v,
                                  sem_in)
        cp_step = pltpu.async_copy(step_hbm, step_v, sem_in)
        cp_ids.wait()

        copies = []
        for j in range(n_chunks):
            for i in range(IDX_CHUNK // L):
                ids16 = ids_v[pl.ds(j * IDX_CHUNK + i * L, L)]
                lo = jnp.bitwise_and(ids16, jnp.int32(0xFFFF))
                pf = (ids16 - lo).astype(jnp.float32)
                k = (pf * jnp.float32(1e-6)).astype(jnp.int32)
                t = pf - k.astype(jnp.float32) * jnp.float32(H)
                r0 = t + lo.astype(jnp.float32)
                r = jnp.where(r0 >= jnp.float32(H), r0 - jnp.float32(H), r0)
                q_v[j, pl.ds(i * L, L)] = r.astype(jnp.int32)
            copies.append(pltpu.async_copy(
                a_hbm.at[q_v.at[j]], a_v.at[pl.ds(j * IDX_CHUNK, IDX_CHUNK)],
                sem_a))
            copies.append(pltpu.async_copy(
                b_hbm.at[q_v.at[j]], b_v.at[pl.ds(j * IDX_CHUNK, IDX_CHUNK)],
                sem_b))
        cp_step.wait()
        step16 = step_v[...]

        out_copies = []
        for j in range(n_chunks):
            copies[2 * j].wait()
            copies[2 * j + 1].wait()
            for i in range(IDX_CHUNK // L):
                a16 = a_v[pl.ds(j * IDX_CHUNK + i * L, L)]
                b16 = b_v[pl.ds(j * IDX_CHUNK + i * L, L)]
                denom = (1.0 - ALPHA) * b16 + ALPHA * (step16 - a16)
                o_v[pl.ds(j * IDX_CHUNK + i * L, L)] = 1.0 / (denom + 1e-8)
            out_copies.append(pltpu.async_copy(
                o_v.at[pl.ds(j * IDX_CHUNK, IDX_CHUNK)],
                out_hbm.at[pl.ds(base + j * IDX_CHUNK, IDX_CHUNK)], sem_in))
        for cp in out_copies:
            cp.wait()

    return sc_kernel


def kernel(batch_item_ids, A, B, step):
    batch = batch_item_ids.shape[0]
    info = plsc.get_sparse_core_info()
    num_cores = 1
    num_workers = num_cores * info.num_subcores
    step_vec = jnp.full((L,), step, dtype=jnp.float32)
    sc = _make_sc_kernel(batch, num_workers, num_cores)
    return sc(batch_item_ids, A, B, step_vec)


# speedup vs baseline: 1.0334x; 1.0334x over previous
import functools

import jax
import jax.numpy as jnp
from jax import lax
from jax.experimental import pallas as pl
from jax.experimental.pallas import tpu as pltpu
from jax.experimental.pallas import tpu_sc as plsc

H = 1_000_000
ALPHA = 0.01
L = 16
IDX_CHUNK = 128


def _make_sc_kernel(batch, num_workers, num_cores):
    n_per_w = batch // num_workers
    n_chunks = n_per_w // IDX_CHUNK
    mesh = plsc.VectorSubcoreMesh(core_axis_name="c", subcore_axis_name="s",
                                  num_cores=num_cores)

    @functools.partial(
        pl.kernel,
        mesh=mesh,
        out_type=jax.ShapeDtypeStruct((batch,), jnp.float32),
        scratch_types=[
            pltpu.VMEM((n_per_w,), jnp.int32),
            pltpu.VMEM((n_chunks, IDX_CHUNK), jnp.int32),
            pltpu.VMEM((n_per_w,), jnp.float32),
            pltpu.VMEM((n_per_w,), jnp.float32),
            pltpu.VMEM((L,), jnp.float32),
            pltpu.VMEM((n_per_w,), jnp.float32),
            pltpu.SemaphoreType.DMA,
            pltpu.SemaphoreType.DMA,
            pltpu.SemaphoreType.DMA,
        ],
    )
    def sc_kernel(ids_hbm, a_hbm, b_hbm, step_hbm, out_hbm,
                  ids_v, q_v, a_v, b_v, step_v, o_v, sem_a, sem_b, sem_in):
        wid = lax.axis_index("s") * num_cores + lax.axis_index("c")
        base = wid * n_per_w

        cp_ids = pltpu.async_copy(ids_hbm.at[pl.ds(base, n_per_w)], ids_v,
                                  sem_in)
        cp_step = pltpu.async_copy(step_hbm, step_v, sem_in)
        cp_ids.wait()

        def fire_chunk(j, carry):
            for i in range(IDX_CHUNK // L):
                ids16 = ids_v[pl.ds(j * IDX_CHUNK + i * L, L)]
                lo = jnp.bitwise_and(ids16, jnp.int32(0xFFFF))
                pf = (ids16 - lo).astype(jnp.float32)
                k = (pf * jnp.float32(1e-6)).astype(jnp.int32)
                t = pf - k.astype(jnp.float32) * jnp.float32(H)
                r0 = t + lo.astype(jnp.float32)
                r = jnp.where(r0 >= jnp.float32(H), r0 - jnp.float32(H), r0)
                q_v[j, pl.ds(i * L, L)] = r.astype(jnp.int32)
            pltpu.async_copy(
                a_hbm.at[q_v.at[j]], a_v.at[pl.ds(j * IDX_CHUNK, IDX_CHUNK)],
                sem_a)
            pltpu.async_copy(
                b_hbm.at[q_v.at[j]], b_v.at[pl.ds(j * IDX_CHUNK, IDX_CHUNK)],
                sem_b)
            return carry

        lax.fori_loop(0, n_chunks, fire_chunk, 0)
        cp_step.wait()
        step16 = step_v[...]

        def drain_chunk(j, carry):
            pltpu.make_async_copy(
                a_hbm.at[q_v.at[j]], a_v.at[pl.ds(j * IDX_CHUNK, IDX_CHUNK)],
                sem_a).wait()
            pltpu.make_async_copy(
                b_hbm.at[q_v.at[j]], b_v.at[pl.ds(j * IDX_CHUNK, IDX_CHUNK)],
                sem_b).wait()

            for i in range(IDX_CHUNK // L):
                a16 = a_v[pl.ds(j * IDX_CHUNK + i * L, L)]
                b16 = b_v[pl.ds(j * IDX_CHUNK + i * L, L)]
                denom = (1.0 - ALPHA) * b16 + ALPHA * (step16 - a16)
                o_v[pl.ds(j * IDX_CHUNK + i * L, L)] = 1.0 / (denom + 1e-8)
            pltpu.async_copy(
                o_v.at[pl.ds(j * IDX_CHUNK, IDX_CHUNK)],
                out_hbm.at[pl.ds(base + j * IDX_CHUNK, IDX_CHUNK)], sem_in)
            return carry

        lax.fori_loop(0, n_chunks, drain_chunk, 0)
        pltpu.make_async_copy(o_v, out_hbm.at[pl.ds(base, n_per_w)],
                              sem_in).wait()

    return sc_kernel


def kernel(batch_item_ids, A, B, step):
    batch = batch_item_ids.shape[0]
    info = plsc.get_sparse_core_info()
    num_cores = info.num_cores
    num_workers = num_cores * info.num_subcores
    step_vec = jnp.full((L,), step, dtype=jnp.float32)
    sc = _make_sc_kernel(batch, num_workers, num_cores)
    return sc(batch_item_ids, A, B, step_vec)


# speedup vs baseline: 1.0407x; 1.0071x over previous
import functools

import jax
import jax.numpy as jnp
from jax import lax
from jax.experimental import pallas as pl
from jax.experimental.pallas import tpu as pltpu
from jax.experimental.pallas import tpu_sc as plsc

H = 1_000_000
ALPHA = 0.01
L = 16
IDX_CHUNK = 128


def _make_sc_kernel(batch, num_workers, num_cores):
    n_per_w = batch // num_workers
    n_chunks = n_per_w // IDX_CHUNK
    mesh = plsc.VectorSubcoreMesh(core_axis_name="c", subcore_axis_name="s",
                                  num_cores=num_cores)

    @functools.partial(
        pl.kernel,
        mesh=mesh,
        out_type=jax.ShapeDtypeStruct((batch,), jnp.float32),
        scratch_types=[
            pltpu.VMEM((n_per_w,), jnp.int32),
            pltpu.VMEM((n_chunks, IDX_CHUNK), jnp.int32),
            pltpu.VMEM((n_per_w,), jnp.float32),
            pltpu.VMEM((n_per_w,), jnp.float32),
            pltpu.VMEM((L,), jnp.float32),
            pltpu.VMEM((n_per_w,), jnp.float32),
            pltpu.SemaphoreType.DMA,
            pltpu.SemaphoreType.DMA,
            pltpu.SemaphoreType.DMA,
        ],
    )
    def sc_kernel(ids_hbm, a_hbm, b_hbm, step_hbm, out_hbm,
                  ids_v, q_v, a_v, b_v, step_v, o_v, sem_a, sem_b, sem_in):
        wid = lax.axis_index("s") * num_cores + lax.axis_index("c")
        base = wid * n_per_w

        cp_ids = pltpu.async_copy(ids_hbm.at[pl.ds(base, n_per_w)], ids_v,
                                  sem_in)
        cp_step = pltpu.async_copy(step_hbm, step_v, sem_in)
        cp_ids.wait()

        def fire_chunk(j, carry):
            def mod16(i, c):
                ids16 = ids_v[pl.ds(j * IDX_CHUNK + i * L, L)]
                lo = jnp.bitwise_and(ids16, jnp.int32(0xFFFF))
                pf = (ids16 - lo).astype(jnp.float32)
                k = (pf * jnp.float32(1e-6)).astype(jnp.int32)
                t = pf - k.astype(jnp.float32) * jnp.float32(H)
                r0 = t + lo.astype(jnp.float32)
                r = jnp.where(r0 >= jnp.float32(H), r0 - jnp.float32(H), r0)
                q_v[j, pl.ds(i * L, L)] = r.astype(jnp.int32)
                return c
            lax.fori_loop(0, IDX_CHUNK // L, mod16, 0)
            pltpu.async_copy(
                a_hbm.at[q_v.at[j]], a_v.at[pl.ds(j * IDX_CHUNK, IDX_CHUNK)],
                sem_a)
            pltpu.async_copy(
                b_hbm.at[q_v.at[j]], b_v.at[pl.ds(j * IDX_CHUNK, IDX_CHUNK)],
                sem_b)
            return carry

        lax.fori_loop(0, n_chunks, fire_chunk, 0)
        cp_step.wait()
        step16 = step_v[...]

        def drain_chunk(j, carry):
            pltpu.make_async_copy(
                a_hbm.at[q_v.at[j]], a_v.at[pl.ds(j * IDX_CHUNK, IDX_CHUNK)],
                sem_a).wait()
            pltpu.make_async_copy(
                b_hbm.at[q_v.at[j]], b_v.at[pl.ds(j * IDX_CHUNK, IDX_CHUNK)],
                sem_b).wait()

            def comb16(i, c):
                a16 = a_v[pl.ds(j * IDX_CHUNK + i * L, L)]
                b16 = b_v[pl.ds(j * IDX_CHUNK + i * L, L)]
                denom = (1.0 - ALPHA) * b16 + ALPHA * (step16 - a16)
                o_v[pl.ds(j * IDX_CHUNK + i * L, L)] = 1.0 / (denom + 1e-8)
                return c
            lax.fori_loop(0, IDX_CHUNK // L, comb16, 0)
            pltpu.async_copy(
                o_v.at[pl.ds(j * IDX_CHUNK, IDX_CHUNK)],
                out_hbm.at[pl.ds(base + j * IDX_CHUNK, IDX_CHUNK)], sem_in)
            return carry

        lax.fori_loop(0, n_chunks, drain_chunk, 0)
        pltpu.make_async_copy(o_v, out_hbm.at[pl.ds(base, n_per_w)],
                              sem_in).wait()

    return sc_kernel


def kernel(batch_item_ids, A, B, step):
    batch = batch_item_ids.shape[0]
    info = plsc.get_sparse_core_info()
    num_cores = info.num_cores
    num_workers = num_cores * info.num_subcores
    step_vec = jnp.full((L,), step, dtype=jnp.float32)
    sc = _make_sc_kernel(batch, num_workers, num_cores)
    return sc(batch_item_ids, A, B, step_vec)


# speedup vs baseline: 1.0407x; 1.0000x over previous
import functools

import jax
import jax.numpy as jnp
from jax import lax
from jax.experimental import pallas as pl
from jax.experimental.pallas import tpu as pltpu
from jax.experimental.pallas import tpu_sc as plsc

H = 1_000_000
ALPHA = 0.01
L = 16
IDX_CHUNK = 128


def _make_sc_kernel(batch, num_workers, num_cores):
    n_per_w = batch // num_workers
    n_chunks = n_per_w // IDX_CHUNK
    mesh = plsc.VectorSubcoreMesh(core_axis_name="c", subcore_axis_name="s",
                                  num_cores=num_cores)

    @functools.partial(
        pl.kernel,
        mesh=mesh,
        out_type=jax.ShapeDtypeStruct((batch,), jnp.float32),
        scratch_types=[
            pltpu.VMEM((n_chunks, IDX_CHUNK), jnp.int32),
            pltpu.VMEM((n_per_w,), jnp.float32),
            pltpu.VMEM((n_per_w,), jnp.float32),
            pltpu.VMEM((L,), jnp.float32),
            pltpu.SemaphoreType.DMA,
            pltpu.SemaphoreType.DMA,
            pltpu.SemaphoreType.DMA,
        ],
    )
    def sc_kernel(ids_hbm, a_hbm, b_hbm, step_hbm, out_hbm,
                  q_v, a_v, b_v, step_v, sem_a, sem_b, sem_io):
        wid = lax.axis_index("s") * num_cores + lax.axis_index("c")
        base = wid * n_per_w

        for j in range(n_chunks):
            pltpu.async_copy(
                ids_hbm.at[pl.ds(base + j * IDX_CHUNK, IDX_CHUNK)],
                q_v.at[j], sem_io)
        cp_step = pltpu.async_copy(step_hbm, step_v, sem_io)

        def fire_chunk(j, carry):
            pltpu.make_async_copy(
                ids_hbm.at[pl.ds(base + j * IDX_CHUNK, IDX_CHUNK)],
                q_v.at[j], sem_io).wait()

            def mod16(i, c):
                ids16 = q_v[j, pl.ds(i * L, L)]
                lo = jnp.bitwise_and(ids16, jnp.int32(0xFFFF))
                pf = (ids16 - lo).astype(jnp.float32)
                k = (pf * jnp.float32(1e-6)).astype(jnp.int32)
                t = pf - k.astype(jnp.float32) * jnp.float32(H)
                r0 = t + lo.astype(jnp.float32)
                r = jnp.where(r0 >= jnp.float32(H), r0 - jnp.float32(H), r0)
                q_v[j, pl.ds(i * L, L)] = r.astype(jnp.int32)
                return c
            lax.fori_loop(0, IDX_CHUNK // L, mod16, 0)
            pltpu.async_copy(
                a_hbm.at[q_v.at[j]], a_v.at[pl.ds(j * IDX_CHUNK, IDX_CHUNK)],
                sem_a)
            pltpu.async_copy(
                b_hbm.at[q_v.at[j]], b_v.at[pl.ds(j * IDX_CHUNK, IDX_CHUNK)],
                sem_b)
            return carry

        lax.fori_loop(0, n_chunks, fire_chunk, 0)
        cp_step.wait()
        step16 = step_v[...]

        def drain_chunk(j, carry):
            pltpu.make_async_copy(
                a_hbm.at[q_v.at[j]], a_v.at[pl.ds(j * IDX_CHUNK, IDX_CHUNK)],
                sem_a).wait()
            pltpu.make_async_copy(
                b_hbm.at[q_v.at[j]], b_v.at[pl.ds(j * IDX_CHUNK, IDX_CHUNK)],
                sem_b).wait()

            def comb16(i, c):
                a16 = a_v[pl.ds(j * IDX_CHUNK + i * L, L)]
                b16 = b_v[pl.ds(j * IDX_CHUNK + i * L, L)]
                denom = (1.0 - ALPHA) * b16 + ALPHA * (step16 - a16)
                b_v[pl.ds(j * IDX_CHUNK + i * L, L)] = 1.0 / (denom + 1e-8)
                return c
            lax.fori_loop(0, IDX_CHUNK // L, comb16, 0)
            pltpu.async_copy(
                b_v.at[pl.ds(j * IDX_CHUNK, IDX_CHUNK)],
                out_hbm.at[pl.ds(base + j * IDX_CHUNK, IDX_CHUNK)], sem_io)
            return carry

        lax.fori_loop(0, n_chunks, drain_chunk, 0)
        pltpu.make_async_copy(b_v, out_hbm.at[pl.ds(base, n_per_w)],
                              sem_io).wait()

    return sc_kernel


def kernel(batch_item_ids, A, B, step):
    batch = batch_item_ids.shape[0]
    info = plsc.get_sparse_core_info()
    num_cores = info.num_cores
    num_workers = num_cores * info.num_subcores
    step_vec = jnp.full((L,), step, dtype=jnp.float32)
    sc = _make_sc_kernel(batch, num_workers, num_cores)
    return sc(batch_item_ids, A, B, step_vec)


# speedup vs baseline: 1.0458x; 1.0048x over previous
import functools

import jax
import jax.numpy as jnp
from jax import lax
from jax.experimental import pallas as pl
from jax.experimental.pallas import tpu as pltpu
from jax.experimental.pallas import tpu_sc as plsc

H = 1_000_000
ALPHA = 0.01
L = 16
IDX_CHUNK = 128


def _make_sc_kernel(batch, num_workers, num_cores):
    n_per_w = batch // num_workers
    n_chunks = n_per_w // IDX_CHUNK
    mesh = plsc.VectorSubcoreMesh(core_axis_name="c", subcore_axis_name="s",
                                  num_cores=num_cores)

    @functools.partial(
        pl.kernel,
        mesh=mesh,
        out_type=jax.ShapeDtypeStruct((batch,), jnp.float32),
        scratch_types=[
            pltpu.VMEM((n_chunks, IDX_CHUNK), jnp.int32),
            pltpu.VMEM((n_per_w,), jnp.float32),
            pltpu.VMEM((n_per_w,), jnp.float32),
            pltpu.VMEM((L,), jnp.float32),
            pltpu.SemaphoreType.DMA,
            pltpu.SemaphoreType.DMA,
            pltpu.SemaphoreType.DMA,
        ],
    )
    def sc_kernel(ids_hbm, a_hbm, b_hbm, step_hbm, out_hbm,
                  q_v, a_v, b_v, step_v, sem_a, sem_b, sem_io):
        wid = lax.axis_index("s") * num_cores + lax.axis_index("c")
        base = wid * n_per_w

        for j in range(n_chunks):
            pltpu.async_copy(
                ids_hbm.at[pl.ds(base + j * IDX_CHUNK, IDX_CHUNK)],
                q_v.at[j], sem_io)
        cp_step = pltpu.async_copy(step_hbm, step_v.at[pl.ds(0, 1)], sem_io)

        def fire_chunk(j, carry):
            pltpu.make_async_copy(
                ids_hbm.at[pl.ds(base + j * IDX_CHUNK, IDX_CHUNK)],
                q_v.at[j], sem_io).wait()

            def mod16(i, c):
                ids16 = q_v[j, pl.ds(i * L, L)]
                lo = jnp.bitwise_and(ids16, jnp.int32(0xFFFF))
                pf = (ids16 - lo).astype(jnp.float32)
                k = (pf * jnp.float32(1e-6)).astype(jnp.int32)
                t = pf - k.astype(jnp.float32) * jnp.float32(H)
                r0 = t + lo.astype(jnp.float32)
                r = jnp.where(r0 >= jnp.float32(H), r0 - jnp.float32(H), r0)
                q_v[j, pl.ds(i * L, L)] = r.astype(jnp.int32)
                return c
            lax.fori_loop(0, IDX_CHUNK // L, mod16, 0)
            pltpu.async_copy(
                a_hbm.at[q_v.at[j]], a_v.at[pl.ds(j * IDX_CHUNK, IDX_CHUNK)],
                sem_a)
            pltpu.async_copy(
                b_hbm.at[q_v.at[j]], b_v.at[pl.ds(j * IDX_CHUNK, IDX_CHUNK)],
                sem_b)
            return carry

        lax.fori_loop(0, n_chunks, fire_chunk, 0)
        cp_step.wait()
        step16 = lax.gather(
            step_v[...], jnp.zeros((L, 1), jnp.int32),
            lax.GatherDimensionNumbers(offset_dims=(),
                                       collapsed_slice_dims=(0,),
                                       start_index_map=(0,)),
            slice_sizes=(1,), mode=lax.GatherScatterMode.PROMISE_IN_BOUNDS)

        def drain_chunk(j, carry):
            pltpu.make_async_copy(
                a_hbm.at[q_v.at[j]], a_v.at[pl.ds(j * IDX_CHUNK, IDX_CHUNK)],
                sem_a).wait()
            pltpu.make_async_copy(
                b_hbm.at[q_v.at[j]], b_v.at[pl.ds(j * IDX_CHUNK, IDX_CHUNK)],
                sem_b).wait()

            def comb16(i, c):
                a16 = a_v[pl.ds(j * IDX_CHUNK + i * L, L)]
                b16 = b_v[pl.ds(j * IDX_CHUNK + i * L, L)]
                denom = (1.0 - ALPHA) * b16 + ALPHA * (step16 - a16)
                b_v[pl.ds(j * IDX_CHUNK + i * L, L)] = 1.0 / (denom + 1e-8)
                return c
            lax.fori_loop(0, IDX_CHUNK // L, comb16, 0)
            pltpu.async_copy(
                b_v.at[pl.ds(j * IDX_CHUNK, IDX_CHUNK)],
                out_hbm.at[pl.ds(base + j * IDX_CHUNK, IDX_CHUNK)], sem_io)
            return carry

        lax.fori_loop(0, n_chunks, drain_chunk, 0)
        pltpu.make_async_copy(b_v, out_hbm.at[pl.ds(base, n_per_w)],
                              sem_io).wait()

    return sc_kernel


def kernel(batch_item_ids, A, B, step):
    batch = batch_item_ids.shape[0]
    info = plsc.get_sparse_core_info()
    num_cores = info.num_cores
    num_workers = num_cores * info.num_subcores
    step_vec = jnp.asarray(step, dtype=jnp.float32).reshape(1)
    sc = _make_sc_kernel(batch, num_workers, num_cores)
    return sc(batch_item_ids, A, B, step_vec)


# speedup vs baseline: 1.0576x; 1.0113x over previous
import functools

import jax
import jax.numpy as jnp
from jax import lax
from jax.experimental import pallas as pl
from jax.experimental.pallas import tpu as pltpu
from jax.experimental.pallas import tpu_sc as plsc

H = 1_000_000
ALPHA = 0.01
L = 16
IDX_CHUNK = 64


def _make_sc_kernel(batch, num_workers, num_cores):
    n_per_w = batch // num_workers
    n_chunks = n_per_w // IDX_CHUNK
    mesh = plsc.VectorSubcoreMesh(core_axis_name="c", subcore_axis_name="s",
                                  num_cores=num_cores)

    @functools.partial(
        pl.kernel,
        mesh=mesh,
        out_type=jax.ShapeDtypeStruct((batch,), jnp.float32),
        scratch_types=[
            pltpu.VMEM((n_chunks, IDX_CHUNK), jnp.int32),
            pltpu.VMEM((n_per_w,), jnp.float32),
            pltpu.VMEM((n_per_w,), jnp.float32),
            pltpu.VMEM((L,), jnp.float32),
            pltpu.SemaphoreType.DMA,
            pltpu.SemaphoreType.DMA,
            pltpu.SemaphoreType.DMA,
        ],
    )
    def sc_kernel(ids_hbm, a_hbm, b_hbm, step_hbm, out_hbm,
                  q_v, a_v, b_v, step_v, sem_a, sem_b, sem_io):
        wid = lax.axis_index("s") * num_cores + lax.axis_index("c")
        base = wid * n_per_w

        for j in range(n_chunks):
            pltpu.async_copy(
                ids_hbm.at[pl.ds(base + j * IDX_CHUNK, IDX_CHUNK)],
                q_v.at[j], sem_io)
        cp_step = pltpu.async_copy(step_hbm, step_v.at[pl.ds(0, 1)], sem_io)

        def fire_chunk(j, carry):
            pltpu.make_async_copy(
                ids_hbm.at[pl.ds(base + j * IDX_CHUNK, IDX_CHUNK)],
                q_v.at[j], sem_io).wait()

            def mod16(i, c):
                ids16 = q_v[j, pl.ds(i * L, L)]
                lo = jnp.bitwise_and(ids16, jnp.int32(0xFFFF))
                pf = (ids16 - lo).astype(jnp.float32)
                k = (pf * jnp.float32(1e-6)).astype(jnp.int32)
                t = pf - k.astype(jnp.float32) * jnp.float32(H)
                r0 = t + lo.astype(jnp.float32)
                r = jnp.where(r0 >= jnp.float32(H), r0 - jnp.float32(H), r0)
                q_v[j, pl.ds(i * L, L)] = r.astype(jnp.int32)
                return c
            lax.fori_loop(0, IDX_CHUNK // L, mod16, 0)
            pltpu.async_copy(
                a_hbm.at[q_v.at[j]], a_v.at[pl.ds(j * IDX_CHUNK, IDX_CHUNK)],
                sem_a)
            pltpu.async_copy(
                b_hbm.at[q_v.at[j]], b_v.at[pl.ds(j * IDX_CHUNK, IDX_CHUNK)],
                sem_b)
            return carry

        lax.fori_loop(0, n_chunks, fire_chunk, 0)
        cp_step.wait()
        step16 = lax.gather(
            step_v[...], jnp.zeros((L, 1), jnp.int32),
            lax.GatherDimensionNumbers(offset_dims=(),
                                       collapsed_slice_dims=(0,),
                                       start_index_map=(0,)),
            slice_sizes=(1,), mode=lax.GatherScatterMode.PROMISE_IN_BOUNDS)

        def drain_chunk(j, carry):
            pltpu.make_async_copy(
                a_hbm.at[q_v.at[j]], a_v.at[pl.ds(j * IDX_CHUNK, IDX_CHUNK)],
                sem_a).wait()
            pltpu.make_async_copy(
                b_hbm.at[q_v.at[j]], b_v.at[pl.ds(j * IDX_CHUNK, IDX_CHUNK)],
                sem_b).wait()

            def comb16(i, c):
                a16 = a_v[pl.ds(j * IDX_CHUNK + i * L, L)]
                b16 = b_v[pl.ds(j * IDX_CHUNK + i * L, L)]
                denom = (1.0 - ALPHA) * b16 + ALPHA * (step16 - a16)
                b_v[pl.ds(j * IDX_CHUNK + i * L, L)] = 1.0 / (denom + 1e-8)
                return c
            lax.fori_loop(0, IDX_CHUNK // L, comb16, 0)
            pltpu.async_copy(
                b_v.at[pl.ds(j * IDX_CHUNK, IDX_CHUNK)],
                out_hbm.at[pl.ds(base + j * IDX_CHUNK, IDX_CHUNK)], sem_io)
            return carry

        lax.fori_loop(0, n_chunks, drain_chunk, 0)
        pltpu.make_async_copy(b_v, out_hbm.at[pl.ds(base, n_per_w)],
                              sem_io).wait()

    return sc_kernel


def kernel(batch_item_ids, A, B, step):
    batch = batch_item_ids.shape[0]
    info = plsc.get_sparse_core_info()
    num_cores = info.num_cores
    num_workers = num_cores * info.num_subcores
    step_vec = jnp.asarray(step, dtype=jnp.float32).reshape(1)
    sc = _make_sc_kernel(batch, num_workers, num_cores)
    return sc(batch_item_ids, A, B, step_vec)
